# Initial kernel scaffold; baseline (speedup 1.0000x reference)
import jax
import jax.numpy as jnp
from jax.experimental import pallas as pl


def kernel(x, pos, sampled_idx, n_sampling, W1, b1, g1, be1, rm1, rv1, W2, b2, g2, be2, rm2, rv2):
    raise NotImplementedError("write your pallas kernel here")


# speedup vs baseline: 13.5970x; 13.5970x over previous
import functools

import jax
import jax.numpy as jnp
from jax import lax
from jax.experimental import pallas as pl
from jax.experimental.pallas import tpu as pltpu
from jax.experimental.pallas import tpu_sc as plsc

EPS = 1e-5
KNN = 16
F32 = jnp.float32


def _sc_gather(table, idx, chunk=256):
    V, D = table.shape
    B = idx.shape[0]
    info = plsc.get_sparse_core_info()
    nw = info.num_cores * info.num_subcores
    bpw = B // nw
    ch = min(bpw, chunk)
    nch = bpw // ch
    mesh = plsc.VectorSubcoreMesh(core_axis_name="c", subcore_axis_name="s")

    @functools.partial(
        pl.kernel,
        mesh=mesh,
        out_type=jax.ShapeDtypeStruct((B, D), F32),
        scratch_types=[
            pltpu.VMEM((ch,), jnp.int32),
            pltpu.VMEM((ch, D), F32),
            pltpu.SemaphoreType.DMA,
        ],
    )
    def k(table_hbm, idx_hbm, out_hbm, idx_v, rows_v, sem):
        wid = lax.axis_index("s") * info.num_cores + lax.axis_index("c")

        def body(i, carry):
            base = wid * bpw + i * ch
            pltpu.sync_copy(idx_hbm.at[pl.ds(base, ch)], idx_v)
            pltpu.async_copy(table_hbm.at[idx_v], rows_v, sem).wait()
            pltpu.sync_copy(rows_v, out_hbm.at[pl.ds(base, ch)])
            return carry

        if nch == 1:
            body(0, 0)
        else:
            lax.fori_loop(0, nch, body, 0)

    return k(table, idx)


def _topk_body(pos_ref, c_ref, out_ref, *, rm, n, c0):
    b = pl.program_id(0)
    p = pos_ref[0]
    cs = c_ref[0]
    d = (cs[:, c0:c0 + 1] - p[0:1, :]) ** 2
    d = d + (cs[:, c0 + 1:c0 + 2] - p[1:2, :]) ** 2
    d = d + (cs[:, c0 + 2:c0 + 3] - p[2:3, :]) ** 2
    iota = lax.broadcasted_iota(jnp.int32, (rm, n), 1)
    big = jnp.float32(jnp.inf)
    cols = []
    for _ in range(KNN):
        mn = jnp.min(d, axis=1, keepdims=True)
        ij = jnp.min(jnp.where(d == mn, iota, n), axis=1, keepdims=True)
        cols.append(ij)
        d = jnp.where(iota == ij, big, d)
    out_ref[0] = jnp.concatenate(cols, axis=1) + b * n


def _topk(pos_t, centers, c0, rm=256):
    bz, _, n = pos_t.shape
    m, w = centers.shape[1], centers.shape[2]
    grid = (bz, m // rm)
    return pl.pallas_call(
        functools.partial(_topk_body, rm=rm, n=n, c0=c0),
        grid=grid,
        in_specs=[
            pl.BlockSpec((1, 3, n), lambda b, r: (b, 0, 0)),
            pl.BlockSpec((1, rm, w), lambda b, r: (b, r, 0)),
        ],
        out_specs=pl.BlockSpec((1, rm, 16), lambda b, r: (b, r, 0)),
        out_shape=jax.ShapeDtypeStruct((bz, m, KNN), jnp.int32),
    )(pos_t, centers)


def _mlp_body(g_ref, c_ref, w1a_ref, w1b_ref, b1_ref, w2_ref, b2_ref,
              out_ref, *, rc, d_in, d_out):
    gx = g_ref[:, :d_in]
    gp = g_ref[:, d_in:d_in + 8]
    c8 = c_ref[:, d_in:d_in + 8]
    w1b = w1b_ref[...]
    h = (jnp.dot(gx, w1a_ref[...], preferred_element_type=F32)
         + jnp.dot(gp, w1b, preferred_element_type=F32)
         + b1_ref[...])
    cc = jnp.dot(c8, w1b, preferred_element_type=F32)
    h = h.reshape(rc, KNN, d_out) - cc[:, None, :]
    h = jnp.maximum(h, 0.0).reshape(rc * KNN, d_out)
    z = jnp.dot(h, w2_ref[...], preferred_element_type=F32) + b2_ref[...]
    out_ref[...] = jnp.max(z.reshape(rc, KNN, d_out), axis=1)


def _mlp_pool(g, centers, w1a, w1b8, b1f, w2f, b2f, rc=256):
    bm = centers.shape[0]
    d_in = w1a.shape[0]
    d_out = w1a.shape[1]
    grid = (bm // rc,)
    return pl.pallas_call(
        functools.partial(_mlp_body, rc=rc, d_in=d_in, d_out=d_out),
        grid=grid,
        in_specs=[
            pl.BlockSpec((rc * KNN, g.shape[1]), lambda r: (r, 0)),
            pl.BlockSpec((rc, centers.shape[1]), lambda r: (r, 0)),
            pl.BlockSpec(w1a.shape, lambda r: (0, 0)),
            pl.BlockSpec(w1b8.shape, lambda r: (0, 0)),
            pl.BlockSpec(b1f.shape, lambda r: (0, 0)),
            pl.BlockSpec(w2f.shape, lambda r: (0, 0)),
            pl.BlockSpec(b2f.shape, lambda r: (0, 0)),
        ],
        out_specs=pl.BlockSpec((rc, d_out), lambda r: (r, 0)),
        out_shape=jax.ShapeDtypeStruct((bm, d_out), F32),
    )(g, centers, w1a, w1b8, b1f, w2f, b2f)


def kernel(x, pos, sampled_idx, n_sampling, W1, b1, g1, be1, rm1, rv1,
           W2, b2, g2, be2, rm2, rv2):
    bz, N, d_in = x.shape
    M = sampled_idx.shape[0] // bz
    d_out = W1.shape[1]

    s1 = g1 / jnp.sqrt(rv1 + EPS)
    w1f = W1 * s1
    b1f = ((b1 - rm1) * s1 + be1).reshape(1, d_out)
    s2 = g2 / jnp.sqrt(rv2 + EPS)
    w2f = W2 * s2
    b2f = ((b2 - rm2) * s2 + be2).reshape(1, d_out)
    w1a = w1f[:d_in]
    w1b8 = jnp.zeros((8, d_out), F32).at[:3].set(w1f[d_in:d_in + 3])

    tw = 2 * 128
    table = jnp.zeros((bz * N, tw), F32)
    table = table.at[:, :d_in].set(x.reshape(bz * N, d_in))
    table = table.at[:, d_in:d_in + 3].set(pos.reshape(bz * N, 3))

    centers = _sc_gather(table, sampled_idx)
    pos_t = pos.transpose(0, 2, 1)
    idxg = _topk(pos_t, centers.reshape(bz, M, tw), d_in)
    g = _sc_gather(table, idxg.reshape(-1))
    out = _mlp_pool(g, centers, w1a, w1b8, b1f, w2f, b2f)
    return out.reshape(bz, M, d_out)


# speedup vs baseline: 15.9213x; 1.1709x over previous
import functools

import jax
import jax.numpy as jnp
from jax import lax
from jax.experimental import pallas as pl
from jax.experimental.pallas import tpu as pltpu
from jax.experimental.pallas import tpu_sc as plsc

EPS = 1e-5
KNN = 16
F32 = jnp.float32


def _sc_gather(table, idx, chunk=256):
    V, D = table.shape
    B = idx.shape[0]
    info = plsc.get_sparse_core_info()
    nw = info.num_cores * info.num_subcores
    bpw = B // nw
    ch = min(bpw, chunk)
    nch = bpw // ch
    mesh = plsc.VectorSubcoreMesh(core_axis_name="c", subcore_axis_name="s")

    @functools.partial(
        pl.kernel,
        mesh=mesh,
        out_type=jax.ShapeDtypeStruct((B, D), F32),
        scratch_types=[
            pltpu.VMEM((ch,), jnp.int32),
            pltpu.VMEM((ch, D), F32),
            pltpu.SemaphoreType.DMA,
        ],
    )
    def k(table_hbm, idx_hbm, out_hbm, idx_v, rows_v, sem):
        wid = lax.axis_index("s") * info.num_cores + lax.axis_index("c")

        def body(i, carry):
            base = wid * bpw + i * ch
            pltpu.sync_copy(idx_hbm.at[pl.ds(base, ch)], idx_v)
            pltpu.async_copy(table_hbm.at[idx_v], rows_v, sem).wait()
            pltpu.sync_copy(rows_v, out_hbm.at[pl.ds(base, ch)])
            return carry

        if nch == 1:
            body(0, 0)
        else:
            lax.fori_loop(0, nch, body, 0)

    return k(table, idx)


def _topk_body(pos_ref, c_ref, out_ref, *, rm, n, c0, b0):
    b = pl.program_id(0)
    p8 = pos_ref[0]
    cs = c_ref[0]
    c8 = cs[:, c0:c0 + 8]
    pn2 = jnp.sum(p8 * p8, axis=0, keepdims=True)
    d = pn2 - 2.0 * jnp.dot(c8, p8, preferred_element_type=F32)
    iota = lax.broadcasted_iota(jnp.int32, (rm, n), 1)
    big = jnp.float32(jnp.inf)
    cols = []
    for _ in range(KNN):
        mn = jnp.min(d, axis=1, keepdims=True)
        msk = d == mn
        ij = jnp.min(jnp.where(msk, iota, n), axis=1, keepdims=True)
        cols.append(ij)
        d = jnp.where(msk, big, d)
    out_ref[0] = jnp.concatenate(cols, axis=1) + (b + b0) * n


def _topk(pos8_t, centers, c0, b0, rm=256):
    bz, _, n = pos8_t.shape
    m, w = centers.shape[1], centers.shape[2]
    grid = (bz, m // rm)
    return pl.pallas_call(
        functools.partial(_topk_body, rm=rm, n=n, c0=c0, b0=b0),
        grid=grid,
        in_specs=[
            pl.BlockSpec((1, 8, n), lambda b, r: (b, 0, 0)),
            pl.BlockSpec((1, rm, w), lambda b, r: (b, r, 0)),
        ],
        out_specs=pl.BlockSpec((1, rm, 16), lambda b, r: (b, r, 0)),
        out_shape=jax.ShapeDtypeStruct((bz, m, KNN), jnp.int32),
    )(pos8_t, centers)


def _mlp_body(g_ref, c_ref, w1a_ref, w1b_ref, b1_ref, w2_ref, b2_ref,
              out_ref, *, rc, d_in, d_out):
    gx = g_ref[:, :d_in]
    gp = g_ref[:, d_in:d_in + 8]
    c8 = c_ref[:, d_in:d_in + 8]
    w1b = w1b_ref[...]
    h = (jnp.dot(gx, w1a_ref[...], preferred_element_type=F32)
         + jnp.dot(gp, w1b, preferred_element_type=F32)
         + b1_ref[...])
    cc = jnp.dot(c8, w1b, preferred_element_type=F32)
    h = h.reshape(rc, KNN, d_out) - cc[:, None, :]
    h = jnp.maximum(h, 0.0).reshape(rc * KNN, d_out)
    z = jnp.dot(h, w2_ref[...], preferred_element_type=F32) + b2_ref[...]
    out_ref[...] = jnp.max(z.reshape(rc, KNN, d_out), axis=1)


def _mlp_pool(g, centers, w1a, w1b8, b1f, w2f, b2f, rc=256):
    bm = centers.shape[0]
    d_in = w1a.shape[0]
    d_out = w1a.shape[1]
    grid = (bm // rc,)
    return pl.pallas_call(
        functools.partial(_mlp_body, rc=rc, d_in=d_in, d_out=d_out),
        grid=grid,
        in_specs=[
            pl.BlockSpec((rc * KNN, g.shape[1]), lambda r: (r, 0)),
            pl.BlockSpec((rc, centers.shape[1]), lambda r: (r, 0)),
            pl.BlockSpec(w1a.shape, lambda r: (0, 0)),
            pl.BlockSpec(w1b8.shape, lambda r: (0, 0)),
            pl.BlockSpec(b1f.shape, lambda r: (0, 0)),
            pl.BlockSpec(w2f.shape, lambda r: (0, 0)),
            pl.BlockSpec(b2f.shape, lambda r: (0, 0)),
        ],
        out_specs=pl.BlockSpec((rc, d_out), lambda r: (r, 0)),
        out_shape=jax.ShapeDtypeStruct((bm, d_out), F32),
    )(g, centers, w1a, w1b8, b1f, w2f, b2f)


def kernel(x, pos, sampled_idx, n_sampling, W1, b1, g1, be1, rm1, rv1,
           W2, b2, g2, be2, rm2, rv2):
    bz, N, d_in = x.shape
    M = sampled_idx.shape[0] // bz
    d_out = W1.shape[1]

    s1 = g1 / jnp.sqrt(rv1 + EPS)
    w1f = W1 * s1
    b1f = ((b1 - rm1) * s1 + be1).reshape(1, d_out)
    s2 = g2 / jnp.sqrt(rv2 + EPS)
    w2f = W2 * s2
    b2f = ((b2 - rm2) * s2 + be2).reshape(1, d_out)
    w1a = w1f[:d_in]
    w1b8 = jnp.zeros((8, d_out), F32).at[:3].set(w1f[d_in:d_in + 3])

    tw = 2 * 128
    table = jnp.zeros((bz * N, tw), F32)
    table = table.at[:, :d_in].set(x.reshape(bz * N, d_in))
    table = table.at[:, d_in:d_in + 3].set(pos.reshape(bz * N, 3))

    centers = _sc_gather(table, sampled_idx)
    pos8_t = jnp.zeros((bz, 8, N), F32).at[:, :3].set(pos.transpose(0, 2, 1))
    cen3 = centers.reshape(bz, M, tw)

    nchunk = 2
    cb = bz // nchunk
    outs = []
    for c in range(nchunk):
        sl = slice(c * cb, (c + 1) * cb)
        idxg = _topk(pos8_t[sl], cen3[sl], d_in, c * cb)
        g = _sc_gather(table, idxg.reshape(-1))
        outs.append(_mlp_pool(g, centers[c * cb * M:(c + 1) * cb * M],
                              w1a, w1b8, b1f, w2f, b2f))
    return jnp.concatenate(outs, axis=0).reshape(bz, M, d_out)
